# baseline (speedup 1.0000x reference)
import functools

import jax
import jax.numpy as jnp
from jax.experimental import pallas as pl
from jax.experimental.pallas import tpu as pltpu


def _pick_tile_k(k_dim: int) -> int:
    for cand in (896, 512, 448, 256, 128):
        if k_dim % cand == 0:
            return cand
    return k_dim


def _body(x_ref, w1_ref, b1_ref, w2_ref, b2_ref, wc_ref, bc_ref, wb_ref,
          bb_ref, score_ref, bbox_ref, acc_ref, *, nk, tile_n):
    k = pl.program_id(0)
    i = pl.program_id(1)
    rows = pl.ds(i * tile_n, tile_n)

    part = jnp.dot(x_ref[...], w1_ref[...], preferred_element_type=jnp.float32)

    @pl.when(k == 0)
    def _init():
        acc_ref[rows, :] = part

    @pl.when(k > 0)
    def _accum():
        acc_ref[rows, :] += part

    @pl.when(k == nk - 1)
    def _finish():
        h = jnp.maximum(acc_ref[rows, :] + b1_ref[...], 0.0)
        h = jnp.maximum(
            jnp.dot(h, w2_ref[...], preferred_element_type=jnp.float32)
            + b2_ref[...], 0.0)
        score_ref[...] = (
            jnp.dot(h, wc_ref[...], preferred_element_type=jnp.float32)
            + bc_ref[...])
        bbox_ref[...] = (
            jnp.dot(h, wb_ref[...], preferred_element_type=jnp.float32)
            + bb_ref[...])


def kernel(x, W1, b1, W2, b2, Wc, bc, Wb, bb):
    n, k_dim = x.shape
    mid = W1.shape[1]
    nc = Wc.shape[1]
    nb = Wb.shape[1]

    tile_n = min(1024, n)
    tile_k = _pick_tile_k(k_dim)
    nt = pl.cdiv(n, tile_n)
    nk = k_dim // tile_k

    b1_2 = b1.reshape(1, -1)
    b2_2 = b2.reshape(1, -1)
    bc_2 = bc.reshape(1, -1)
    bb_2 = bb.reshape(1, -1)

    grid = (nk, nt)
    out_shapes = (
        jax.ShapeDtypeStruct((n, nc), jnp.float32),
        jax.ShapeDtypeStruct((n, nb), jnp.float32),
    )

    def out_idx(k, i):
        return (jnp.where(k == nk - 1, i, 0), 0)

    in_specs = [
        pl.BlockSpec((tile_n, tile_k), lambda k, i: (i, k)),
        pl.BlockSpec((tile_k, mid), lambda k, i: (k, 0)),
        pl.BlockSpec((1, mid), lambda k, i: (0, 0)),
        pl.BlockSpec((mid, mid), lambda k, i: (0, 0)),
        pl.BlockSpec((1, mid), lambda k, i: (0, 0)),
        pl.BlockSpec((mid, nc), lambda k, i: (0, 0)),
        pl.BlockSpec((1, nc), lambda k, i: (0, 0)),
        pl.BlockSpec((mid, nb), lambda k, i: (0, 0)),
        pl.BlockSpec((1, nb), lambda k, i: (0, 0)),
    ]
    out_specs = (
        pl.BlockSpec((tile_n, nc), out_idx),
        pl.BlockSpec((tile_n, nb), out_idx),
    )

    return pl.pallas_call(
        functools.partial(_body, nk=nk, tile_n=tile_n),
        grid=grid,
        in_specs=in_specs,
        out_specs=out_specs,
        out_shape=out_shapes,
        scratch_shapes=[pltpu.VMEM((nt * tile_n, mid), jnp.float32)],
        compiler_params=pltpu.CompilerParams(
            dimension_semantics=("arbitrary", "arbitrary"),
        ),
    )(x, W1, b1_2, W2, b2_2, Wc, bc_2, Wb, bb_2)


# speedup vs baseline: 1.2653x; 1.2653x over previous
import functools

import jax
import jax.numpy as jnp
from jax.experimental import pallas as pl
from jax.experimental.pallas import tpu as pltpu

ROW_CHUNK = 1000


def _pick_tile_k(k_dim: int) -> int:
    for cand in (256, 128, 512):
        if k_dim % cand == 0:
            return cand
    return k_dim


def _row_slices(n):
    chunk = ROW_CHUNK if (n % ROW_CHUNK == 0 and (n // ROW_CHUNK) > 0) else n
    return [pl.ds(i * chunk, chunk) for i in range(n // chunk)]


def _body(x_ref, w1_ref, b1_ref, w2_ref, b2_ref, wc_ref, bc_ref, wb_ref,
          bb_ref, score_ref, bbox_ref, acc_ref, *, nk, n):
    k = pl.program_id(0)
    slices = _row_slices(n)
    w1 = w1_ref[...].astype(jnp.bfloat16)

    @pl.when(k == 0)
    def _init():
        for sl in slices:
            acc_ref[sl, :] = jnp.dot(x_ref[sl, :].astype(jnp.bfloat16), w1,
                                     preferred_element_type=jnp.float32)

    @pl.when(k > 0)
    def _accum():
        for sl in slices:
            acc_ref[sl, :] += jnp.dot(x_ref[sl, :].astype(jnp.bfloat16), w1,
                                      preferred_element_type=jnp.float32)

    @pl.when(k == nk - 1)
    def _finish():
        w2 = w2_ref[...].astype(jnp.bfloat16)
        wc = wc_ref[...].astype(jnp.bfloat16)
        wb = wb_ref[...].astype(jnp.bfloat16)
        for sl in slices:
            h = jnp.maximum(acc_ref[sl, :] + b1_ref[...],
                            0.0).astype(jnp.bfloat16)
            h2 = jnp.maximum(
                jnp.dot(h, w2, preferred_element_type=jnp.float32)
                + b2_ref[...], 0.0).astype(jnp.bfloat16)
            score_ref[sl, :] = (
                jnp.dot(h2, wc, preferred_element_type=jnp.float32)
                + bc_ref[...])
            bbox_ref[sl, :] = (
                jnp.dot(h2, wb, preferred_element_type=jnp.float32)
                + bb_ref[...])


def kernel(x, W1, b1, W2, b2, Wc, bc, Wb, bb):
    n, k_dim = x.shape
    mid = W1.shape[1]
    nc = Wc.shape[1]
    nb = Wb.shape[1]

    tile_k = _pick_tile_k(k_dim)
    nk = k_dim // tile_k

    b1_2 = b1.reshape(1, -1)
    b2_2 = b2.reshape(1, -1)
    bc_2 = bc.reshape(1, -1)
    bb_2 = bb.reshape(1, -1)

    out_shapes = (
        jax.ShapeDtypeStruct((n, nc), jnp.float32),
        jax.ShapeDtypeStruct((n, nb), jnp.float32),
    )
    in_specs = [
        pl.BlockSpec((n, tile_k), lambda k: (0, k)),
        pl.BlockSpec((tile_k, mid), lambda k: (k, 0)),
        pl.BlockSpec((1, mid), lambda k: (0, 0)),
        pl.BlockSpec((mid, mid), lambda k: (0, 0)),
        pl.BlockSpec((1, mid), lambda k: (0, 0)),
        pl.BlockSpec((mid, nc), lambda k: (0, 0)),
        pl.BlockSpec((1, nc), lambda k: (0, 0)),
        pl.BlockSpec((mid, nb), lambda k: (0, 0)),
        pl.BlockSpec((1, nb), lambda k: (0, 0)),
    ]
    out_specs = (
        pl.BlockSpec((n, nc), lambda k: (0, 0)),
        pl.BlockSpec((n, nb), lambda k: (0, 0)),
    )

    return pl.pallas_call(
        functools.partial(_body, nk=nk, n=n),
        grid=(nk,),
        in_specs=in_specs,
        out_specs=out_specs,
        out_shape=out_shapes,
        scratch_shapes=[pltpu.VMEM((n, mid), jnp.float32)],
        compiler_params=pltpu.CompilerParams(
            dimension_semantics=("arbitrary",),
        ),
    )(x, W1, b1_2, W2, b2_2, Wc, bc_2, Wb, bb_2)
